# baseline (speedup 1.0000x reference)
import functools

import jax
import jax.numpy as jnp
from jax import lax
from jax.experimental import pallas as pl
from jax.experimental.pallas import tpu as pltpu
from jax.experimental.pallas import tpu_sc as plsc

NUM_CLASSES = 22
EMB_DIM = 512
SLOT_SIZE = 2048
BATCH = 16384

NC, NS = 2, 16
NW = NC * NS
BPW = BATCH // NW
TROWS = 32
RPT = TROWS // NS


NBLK = 4
BLK = SLOT_SIZE // NBLK


def _mlp_body(embs_ref, w1_ref, b1_ref, w2_ref, b2_ref, out_ref, h_ref):
    @pl.when(pl.program_id(0) == 0)
    def _h():
        e = jnp.concatenate(
            [embs_ref[...], jnp.zeros((TROWS - NUM_CLASSES, EMB_DIM), jnp.float32)],
            axis=0,
        )
        h = jnp.dot(e, w1_ref[...], preferred_element_type=jnp.float32)
        h = h + b1_ref[...]
        h_ref[...] = 0.5 * h * (1.0 + lax.erf(h * 0.7071067811865476))

    out = jnp.dot(h_ref[...], w2_ref[...], preferred_element_type=jnp.float32)
    out_ref[...] = out + b2_ref[...]


def _mlp(embs, W1, b1, W2, b2):
    return pl.pallas_call(
        _mlp_body,
        grid=(NBLK,),
        in_specs=[
            pl.BlockSpec((NUM_CLASSES, EMB_DIM), lambda j: (0, 0)),
            pl.BlockSpec((EMB_DIM, 4 * EMB_DIM), lambda j: (0, 0)),
            pl.BlockSpec((1, 4 * EMB_DIM), lambda j: (0, 0)),
            pl.BlockSpec((4 * EMB_DIM, BLK), lambda j: (0, j)),
            pl.BlockSpec((1, BLK), lambda j: (0, j)),
        ],
        out_specs=pl.BlockSpec((TROWS, BLK), lambda j: (0, j)),
        scratch_shapes=[pltpu.VMEM((TROWS, 4 * EMB_DIM), jnp.float32)],
        out_shape=jax.ShapeDtypeStruct((TROWS, SLOT_SIZE), jnp.float32),
    )(embs, W1, b1.reshape(1, -1), W2, b2.reshape(1, -1))


_mesh = plsc.VectorSubcoreMesh(core_axis_name="c", subcore_axis_name="s")


@functools.partial(
    pl.kernel,
    mesh=_mesh,
    out_type=jax.ShapeDtypeStruct((BATCH, SLOT_SIZE), jnp.float32),
    scratch_types=[
        pltpu.VMEM((BPW,), jnp.int32),
        pltpu.VMEM_SHARED((TROWS, SLOT_SIZE), jnp.float32),
        pltpu.SemaphoreType.DMA,
    ],
)
def _gather(labels_hbm, table_hbm, out_hbm, idx_v, table_s, sem):
    cid = lax.axis_index("c")
    sid = lax.axis_index("s")
    wid = sid * NC + cid
    base = wid * BPW
    pltpu.sync_copy(labels_hbm.at[pl.ds(base, BPW)], idx_v)

    pltpu.sync_copy(
        table_hbm.at[pl.ds(sid * RPT, RPT)], table_s.at[pl.ds(sid * RPT, RPT)]
    )
    plsc.subcore_barrier()

    K = 16

    def outer(w, carry):
        labs = idx_v[pl.ds(w * K, K)]
        for j in range(K):
            pltpu.async_copy(table_s.at[labs[j]], out_hbm.at[base + w * K + j], sem)
        pltpu.make_async_copy(
            table_s.at[pl.ds(0, K)], out_hbm.at[pl.ds(base + w * K, K)], sem
        ).wait()
        return carry

    lax.fori_loop(0, BPW // K, outer, 0, unroll=False)


def kernel(labels, embs, W1, b1, W2, b2):
    proj = _mlp(embs, W1, b1, W2, b2)
    return _gather(labels.astype(jnp.int32), proj)


# speedup vs baseline: 1.3220x; 1.3220x over previous
import functools

import jax
import jax.numpy as jnp
from jax import lax
from jax.experimental import pallas as pl
from jax.experimental.pallas import tpu as pltpu
from jax.experimental.pallas import tpu_sc as plsc

NUM_CLASSES = 22
EMB_DIM = 512
SLOT_SIZE = 2048
BATCH = 16384

NC, NS = 2, 16
NW = NC * NS
BPW = BATCH // NW
TROWS = 32
RPT = TROWS // NS


NBLK = 4
BLK = SLOT_SIZE // NBLK


def _mlp_body(embs_ref, w1_ref, b1_ref, w2_ref, b2_ref, out_ref, h_ref):
    @pl.when(pl.program_id(0) == 0)
    def _h():
        e = jnp.concatenate(
            [embs_ref[...], jnp.zeros((TROWS - NUM_CLASSES, EMB_DIM), jnp.float32)],
            axis=0,
        )
        h = jnp.dot(e, w1_ref[...], preferred_element_type=jnp.float32)
        h = h + b1_ref[...]
        h_ref[...] = 0.5 * h * (1.0 + lax.erf(h * 0.7071067811865476))

    out = jnp.dot(h_ref[...], w2_ref[...], preferred_element_type=jnp.float32)
    out_ref[...] = out + b2_ref[...]


def _mlp(embs, W1, b1, W2, b2):
    return pl.pallas_call(
        _mlp_body,
        grid=(NBLK,),
        in_specs=[
            pl.BlockSpec((NUM_CLASSES, EMB_DIM), lambda j: (0, 0)),
            pl.BlockSpec((EMB_DIM, 4 * EMB_DIM), lambda j: (0, 0)),
            pl.BlockSpec((1, 4 * EMB_DIM), lambda j: (0, 0)),
            pl.BlockSpec((4 * EMB_DIM, BLK), lambda j: (0, j)),
            pl.BlockSpec((1, BLK), lambda j: (0, j)),
        ],
        out_specs=pl.BlockSpec((TROWS, BLK), lambda j: (0, j)),
        scratch_shapes=[pltpu.VMEM((TROWS, 4 * EMB_DIM), jnp.float32)],
        out_shape=jax.ShapeDtypeStruct((TROWS, SLOT_SIZE), jnp.float32),
    )(embs, W1, b1.reshape(1, -1), W2, b2.reshape(1, -1))


_mesh = plsc.VectorSubcoreMesh(core_axis_name="c", subcore_axis_name="s")


@functools.partial(
    pl.kernel,
    mesh=_mesh,
    out_type=jax.ShapeDtypeStruct((BATCH, SLOT_SIZE), jnp.float32),
    scratch_types=[
        pltpu.VMEM((BPW,), jnp.int32),
        pltpu.VMEM((TROWS, SLOT_SIZE), jnp.float32),
        pltpu.VMEM_SHARED((TROWS, SLOT_SIZE), jnp.float32),
        pltpu.SemaphoreType.DMA,
    ],
)
def _gather(labels_hbm, table_hbm, out_hbm, idx_v, table_v, table_s, sem):
    cid = lax.axis_index("c")
    sid = lax.axis_index("s")
    wid = sid * NC + cid
    base = wid * BPW
    pltpu.sync_copy(labels_hbm.at[pl.ds(base, BPW)], idx_v)

    pltpu.sync_copy(
        table_hbm.at[pl.ds(sid * RPT, RPT)], table_s.at[pl.ds(sid * RPT, RPT)]
    )
    plsc.subcore_barrier()
    pltpu.sync_copy(table_s, table_v)

    K = 16

    def outer(w, carry):
        labs = idx_v[pl.ds(w * K, K)]
        for j in range(K):
            pltpu.async_copy(table_v.at[labs[j]], out_hbm.at[base + w * K + j], sem)
        pltpu.make_async_copy(
            table_v.at[pl.ds(0, K)], out_hbm.at[pl.ds(base + w * K, K)], sem
        ).wait()
        return carry

    lax.fori_loop(0, BPW // K, outer, 0, unroll=False)


def kernel(labels, embs, W1, b1, W2, b2):
    proj = _mlp(embs, W1, b1, W2, b2)
    return _gather(labels.astype(jnp.int32), proj)


# speedup vs baseline: 1.3298x; 1.0059x over previous
import functools

import jax
import jax.numpy as jnp
from jax import lax
from jax.experimental import pallas as pl
from jax.experimental.pallas import tpu as pltpu
from jax.experimental.pallas import tpu_sc as plsc

NUM_CLASSES = 22
EMB_DIM = 512
SLOT_SIZE = 2048
BATCH = 16384

NC, NS = 2, 16
NW = NC * NS
BPW = BATCH // NW
TROWS = 32
RPT = TROWS // NS


NBLK = 4
BLK = SLOT_SIZE // NBLK


def _mlp_body(embs_ref, w1_ref, b1_ref, w2_ref, b2_ref, out_ref, h_ref):
    @pl.when(pl.program_id(0) == 0)
    def _h():
        e = jnp.concatenate(
            [embs_ref[...], jnp.zeros((TROWS - NUM_CLASSES, EMB_DIM), jnp.float32)],
            axis=0,
        )
        h = jnp.dot(e, w1_ref[...], preferred_element_type=jnp.float32)
        h = h + b1_ref[...]
        h_ref[...] = 0.5 * h * (1.0 + lax.erf(h * 0.7071067811865476))

    out = jnp.dot(h_ref[...], w2_ref[...], preferred_element_type=jnp.float32)
    out_ref[...] = out + b2_ref[...]


def _mlp(embs, W1, b1, W2, b2):
    return pl.pallas_call(
        _mlp_body,
        grid=(NBLK,),
        in_specs=[
            pl.BlockSpec((NUM_CLASSES, EMB_DIM), lambda j: (0, 0)),
            pl.BlockSpec((EMB_DIM, 4 * EMB_DIM), lambda j: (0, 0)),
            pl.BlockSpec((1, 4 * EMB_DIM), lambda j: (0, 0)),
            pl.BlockSpec((4 * EMB_DIM, BLK), lambda j: (0, j)),
            pl.BlockSpec((1, BLK), lambda j: (0, j)),
        ],
        out_specs=pl.BlockSpec((TROWS, BLK), lambda j: (0, j)),
        scratch_shapes=[pltpu.VMEM((TROWS, 4 * EMB_DIM), jnp.float32)],
        out_shape=jax.ShapeDtypeStruct((TROWS, SLOT_SIZE), jnp.float32),
    )(embs, W1, b1.reshape(1, -1), W2, b2.reshape(1, -1))


_mesh = plsc.VectorSubcoreMesh(core_axis_name="c", subcore_axis_name="s")


@functools.partial(
    pl.kernel,
    mesh=_mesh,
    out_type=jax.ShapeDtypeStruct((BATCH, SLOT_SIZE), jnp.float32),
    scratch_types=[
        pltpu.VMEM((BPW,), jnp.int32),
        pltpu.VMEM((TROWS, SLOT_SIZE), jnp.float32),
        pltpu.VMEM_SHARED((TROWS, SLOT_SIZE), jnp.float32),
        pltpu.SemaphoreType.DMA,
        pltpu.SemaphoreType.DMA,
    ],
)
def _gather(labels_hbm, table_hbm, out_hbm, idx_v, table_v, table_s, sem, lsem):
    cid = lax.axis_index("c")
    sid = lax.axis_index("s")
    wid = sid * NC + cid
    base = wid * BPW
    labels_cp = pltpu.async_copy(labels_hbm.at[pl.ds(base, BPW)], idx_v, lsem)

    pltpu.sync_copy(
        table_hbm.at[pl.ds(sid * RPT, RPT)], table_s.at[pl.ds(sid * RPT, RPT)]
    )
    plsc.subcore_barrier()
    pltpu.sync_copy(table_s, table_v)
    labels_cp.wait()

    K = 16

    def outer(w, carry):
        labs = idx_v[pl.ds(w * K, K)]
        for j in range(K):
            pltpu.async_copy(table_v.at[labs[j]], out_hbm.at[base + w * K + j], sem)
        pltpu.make_async_copy(
            table_v.at[pl.ds(0, K)], out_hbm.at[pl.ds(base + w * K, K)], sem
        ).wait()
        return carry

    lax.fori_loop(0, BPW // K, outer, 0, unroll=False)


def kernel(labels, embs, W1, b1, W2, b2):
    proj = _mlp(embs, W1, b1, W2, b2)
    return _gather(labels.astype(jnp.int32), proj)
